# baseline (speedup 1.0000x reference)
import functools
import math

import jax
import jax.numpy as jnp
from jax.experimental import pallas as pl
from jax.experimental.pallas import tpu as pltpu

ROWS, COLS = 16384, 200
BLK = 512
SCALE = 7.0 / math.pi
HALF_PI = math.pi / 2.0


def _body(x_ref, o_ref):
    v = x_ref[...]
    idx = (v * SCALE).astype(jnp.int32)
    o_ref[...] = idx.astype(jnp.float32) * HALF_PI


@jax.jit
def kernel(inputs):
    return pl.pallas_call(
        _body,
        grid=(ROWS // BLK,),
        in_specs=[pl.BlockSpec((BLK, COLS), lambda i: (i, 0))],
        out_specs=pl.BlockSpec((BLK, COLS), lambda i: (i, 0)),
        out_shape=jax.ShapeDtypeStruct((ROWS, COLS), jnp.float32),
    )(inputs)


# speedup vs baseline: 1.2839x; 1.2839x over previous
import functools
import math

import jax
import jax.numpy as jnp
from jax.experimental import pallas as pl
from jax.experimental.pallas import tpu as pltpu

ROWS, COLS = 16384, 200
BLK = 4096
SCALE = 7.0 / math.pi
HALF_PI = math.pi / 2.0


def _body(x_ref, o_ref):
    v = x_ref[...]
    idx = (v * SCALE).astype(jnp.int32)
    o_ref[...] = idx.astype(jnp.float32) * HALF_PI


@jax.jit
def kernel(inputs):
    return pl.pallas_call(
        _body,
        grid=(ROWS // BLK,),
        in_specs=[pl.BlockSpec((BLK, COLS), lambda i: (i, 0))],
        out_specs=pl.BlockSpec((BLK, COLS), lambda i: (i, 0)),
        out_shape=jax.ShapeDtypeStruct((ROWS, COLS), jnp.float32),
    )(inputs)
